# baseline (speedup 1.0000x reference)
import functools

import jax
import jax.numpy as jnp
from jax.experimental import pallas as pl


def _proj_kernel(posT_ref, a_ref, b_ref, m_ref, out_ref):
    pT = posT_ref[0]
    p_cols = jnp.transpose(pT)
    px = p_cols[:, 0:1]
    py = p_cols[:, 1:2]
    pz = p_cols[:, 2:3]
    ax = a_ref[0, 0:1, :]
    ay = a_ref[0, 1:2, :]
    az = a_ref[0, 2:3, :]
    bx = b_ref[0, 0:1, :]
    by = b_ref[0, 1:2, :]
    bz = b_ref[0, 2:3, :]
    m = m_ref[0]

    dx = bx - ax
    dy = by - ay
    dz = bz - az
    dd = jnp.maximum(dx * dx + dy * dy + dz * dz, 1e-12)
    rdd = 1.0 / dd
    mbig = (1.0 - m) * 1e30

    ex = px - ax
    ey = py - ay
    ez = pz - az
    tn = ex * dx + ey * dy + ez * dz
    pa2 = ex * ex + ey * ey + ez * ez
    t = jnp.clip(tn * rdd, 0.0, 1.0)
    dist2 = pa2 + t * (t * dd - tn - tn) + mbig

    T = dist2.shape[0]
    NSP = dist2.shape[1]
    best = jnp.argmin(dist2, axis=1)

    onehotT = (jax.lax.broadcasted_iota(jnp.int32, (NSP, T), 0)
               == best[None, :]).astype(jnp.float32)
    zrow = jnp.zeros_like(ax)
    A8 = jnp.concatenate([ax, ay, az, dx, dy, dz, rdd, zrow], axis=0)
    G = jax.lax.dot_general(
        A8, onehotT,
        dimension_numbers=(((1,), (0,)), ((), ())),
        preferred_element_type=jnp.float32,
        precision=jax.lax.Precision.HIGHEST,
    )

    axb = G[0:1, :]
    ayb = G[1:2, :]
    azb = G[2:3, :]
    dxb = G[3:4, :]
    dyb = G[4:5, :]
    dzb = G[5:6, :]
    rddb = G[6:7, :]
    pxr = pT[0:1, :]
    pyr = pT[1:2, :]
    pzr = pT[2:3, :]
    tb = jnp.clip(((pxr - axb) * dxb + (pyr - ayb) * dyb
                   + (pzr - azb) * dzb) * rddb, 0.0, 1.0)
    qx = axb + tb * dxb
    qy = ayb + tb * dyb
    qz = azb + tb * dzb
    q = jnp.concatenate([qx, qy, qz], axis=0)

    has_valid = jnp.any(m > 0.0)
    out_ref[0] = jnp.where(has_valid, q, pT)


@functools.partial(jax.jit, static_argnames=())
def _run(posT, aT, bT, maskf):
    N = posT.shape[0]
    T = posT.shape[2]
    NSP = aT.shape[2]
    out = pl.pallas_call(
        _proj_kernel,
        grid=(N,),
        in_specs=[
            pl.BlockSpec((1, 3, T), lambda n: (n, 0, 0)),
            pl.BlockSpec((1, 3, NSP), lambda n: (n, 0, 0)),
            pl.BlockSpec((1, 3, NSP), lambda n: (n, 0, 0)),
            pl.BlockSpec((1, 1, NSP), lambda n: (n, 0, 0)),
        ],
        out_specs=pl.BlockSpec((1, 3, T), lambda n: (n, 0, 0)),
        out_shape=jax.ShapeDtypeStruct((N, 3, T), posT.dtype),
    )(posT, aT, bT, maskf)
    return out


def kernel(selected_traj, road_points, road_mask):
    pos = selected_traj[..., 0:3]
    rest = selected_traj[..., 3:]
    N, NB, NP, D = road_points.shape
    NS = NB * (NP - 1)
    NSP = (NS + 127) // 128 * 128

    rpT = road_points.transpose(0, 3, 1, 2)
    aT = rpT[:, :, :, :-1].reshape(N, D, NS)
    bT = rpT[:, :, :, 1:].reshape(N, D, NS)
    seg_mask = (road_mask[:, :, :-1] & road_mask[:, :, 1:]).reshape(N, NS)

    pad = ((0, 0), (0, 0), (0, NSP - NS))
    aT = jnp.pad(aT, pad)
    bT = jnp.pad(bT, pad)
    maskf = jnp.pad(seg_mask.astype(jnp.float32)[:, None, :], pad)
    posT = pos.transpose(0, 2, 1).astype(jnp.float32)

    outT = _run(posT, aT, bT, maskf)
    pos_proj = outT.transpose(0, 2, 1)
    if rest.shape[-1] == 0:
        return pos_proj
    return jnp.concatenate([pos_proj, rest], axis=-1)


# speedup vs baseline: 1.1744x; 1.1744x over previous
import dataclasses
import functools

import jax
import jax.numpy as jnp
from jax import lax
from jax.experimental import pallas as pl
from jax.experimental.pallas import tpu as pltpu
from jax.experimental.pallas import tpu_sc as plsc


def _tc_kernel(posT_ref, a_ref, m_ref, rows_ref, hv_ref):
    pT = posT_ref[0]
    p_cols = jnp.transpose(pT)
    px = p_cols[:, 0:1]
    py = p_cols[:, 1:2]
    pz = p_cols[:, 2:3]
    ax = a_ref[0, 0:1, :]
    ay = a_ref[0, 1:2, :]
    az = a_ref[0, 2:3, :]
    m = m_ref[0]

    z1 = jnp.zeros((1, 1), jnp.float32)
    bx = jnp.concatenate([ax[:, 1:], z1], axis=1)
    by = jnp.concatenate([ay[:, 1:], z1], axis=1)
    bz = jnp.concatenate([az[:, 1:], z1], axis=1)

    dx = bx - ax
    dy = by - ay
    dz = bz - az
    dd = jnp.maximum(dx * dx + dy * dy + dz * dz, 1e-12)
    rdd = 1.0 / dd
    mbig = (1.0 - m) * 1e30

    ex = px - ax
    ey = py - ay
    ez = pz - az
    tn = ex * dx + ey * dy + ez * dz
    pa2 = ex * ex + ey * ey + ez * ez
    t = jnp.clip(tn * rdd, 0.0, 1.0)
    dist2 = pa2 + t * (t * dd - tn - tn) + mbig

    best = jnp.argmin(dist2, axis=1).astype(jnp.int32)
    rows_ref[0] = (best * 3)[None, :]
    hv = jnp.any(m > 0.0).astype(jnp.float32)
    hv_ref[0] = jnp.broadcast_to(hv, (1, pT.shape[1]))


def _tc_run(posT, rpT, maskf):
    N = posT.shape[0]
    T = posT.shape[2]
    NSP = rpT.shape[2]
    return pl.pallas_call(
        _tc_kernel,
        grid=(N,),
        in_specs=[
            pl.BlockSpec((1, 3, T), lambda n: (n, 0, 0)),
            pl.BlockSpec((1, 3, NSP), lambda n: (n, 0, 0)),
            pl.BlockSpec((1, 1, NSP), lambda n: (n, 0, 0)),
        ],
        out_specs=[
            pl.BlockSpec((1, 1, T), lambda n: (n, 0, 0)),
            pl.BlockSpec((1, 1, T), lambda n: (n, 0, 0)),
        ],
        out_shape=[
            jax.ShapeDtypeStruct((N, 1, T), jnp.int32),
            jax.ShapeDtypeStruct((N, 1, T), jnp.float32),
        ],
    )(posT, rpT, maskf)


def _sc_kernel(rp_ref, rows_ref, posT_ref, hv_ref, out_ref,
               table_v, idx_v, pos_v, hvrow_v, out_v):
    wid = lax.axis_index("s") * 2 + lax.axis_index("c")
    for k in range(2):
        n = wid * 2 + k
        pltpu.sync_copy(rp_ref.at[n], table_v)
        pltpu.sync_copy(rows_ref.at[n], idx_v)
        pltpu.sync_copy(posT_ref.at[n], pos_v)
        pltpu.sync_copy(hv_ref.at[n], hvrow_v)
        for c in range(4):
            sl = pl.ds(c * 16, 16)
            r3 = idx_v[0, sl]
            ax = plsc.load_gather(table_v, [r3])
            ay = plsc.load_gather(table_v, [r3 + 1])
            az = plsc.load_gather(table_v, [r3 + 2])
            bx = plsc.load_gather(table_v, [r3 + 3])
            by = plsc.load_gather(table_v, [r3 + 4])
            bz = plsc.load_gather(table_v, [r3 + 5])
            px = pos_v[0, sl]
            py = pos_v[1, sl]
            pz = pos_v[2, sl]
            dx = bx - ax
            dy = by - ay
            dz = bz - az
            dd = jnp.maximum(dx * dx + dy * dy + dz * dz, 1e-12)
            tn = (px - ax) * dx + (py - ay) * dy + (pz - az) * dz
            t = jnp.clip(tn / dd, 0.0, 1.0)
            qx = ax + t * dx
            qy = ay + t * dy
            qz = az + t * dz
            hv = hvrow_v[0, sl] > 0.0
            out_v[0, sl] = jnp.where(hv, qx, px)
            out_v[1, sl] = jnp.where(hv, qy, py)
            out_v[2, sl] = jnp.where(hv, qz, pz)
        pltpu.sync_copy(out_v, out_ref.at[n])


def _sc_run(rp_flat, rows, posT, hvb):
    N = posT.shape[0]
    T = posT.shape[2]
    mesh = plsc.VectorSubcoreMesh(core_axis_name="c", subcore_axis_name="s",
                                  num_cores=2, num_subcores=16)
    cp = pltpu.CompilerParams()
    if "needs_layout_passes" in pltpu.CompilerParams.__dataclass_fields__:
        cp = dataclasses.replace(cp, needs_layout_passes=False)
    k = pl.kernel(
        _sc_kernel,
        out_type=jax.ShapeDtypeStruct((N, 3, T), jnp.float32),
        mesh=mesh,
        scratch_types=[
            pltpu.VMEM((rp_flat.shape[1],), jnp.float32),
            pltpu.VMEM((1, T), jnp.int32),
            pltpu.VMEM((3, T), jnp.float32),
            pltpu.VMEM((1, T), jnp.float32),
            pltpu.VMEM((3, T), jnp.float32),
        ],
        compiler_params=cp,
    )
    return k(rp_flat, rows, posT, hvb)


@jax.jit
def _run(selected_traj, road_points, road_mask):
    pos = selected_traj[..., 0:3]
    rest = selected_traj[..., 3:]
    N, NB, NP, D = road_points.shape
    NSP = NB * NP

    rpT = road_points.transpose(0, 3, 1, 2).reshape(N, D, NSP)
    rp_flat = road_points.reshape(N, NSP * D)
    posT = pos.transpose(0, 2, 1).astype(jnp.float32)

    rm_flat = road_mask.reshape(N, NSP)
    m2d = jnp.pad(rm_flat[:, :-1] & rm_flat[:, 1:], ((0, 0), (0, 1)))
    lane_ok = (jnp.arange(NSP, dtype=jnp.int32) % NP) != (NP - 1)
    maskf = (m2d & lane_ok[None, :]).astype(jnp.float32)[:, None, :]

    rows, hvb = _tc_run(posT, rpT, maskf)
    outT = _sc_run(rp_flat, rows, posT, hvb)
    pos_proj = outT.transpose(0, 2, 1)
    if rest.shape[-1] == 0:
        return pos_proj
    return jnp.concatenate([pos_proj, rest], axis=-1)


def kernel(selected_traj, road_points, road_mask):
    return _run(selected_traj, road_points, road_mask)
